# baseline (speedup 1.0000x reference)
import functools

import jax
import jax.numpy as jnp
from jax import lax
from jax.experimental import pallas as pl
from jax.experimental.pallas import tpu as pltpu
from jax.experimental.pallas import tpu_sc as plsc

HORSE_COUNT = 1000000
NDIM = 16
BATCH = 16384
CTX = 6

NC = 2
NS = 16
NW = NC * NS

CHUNK = 128
ROWS_W = BATCH // NW
CH_W = ROWS_W * 8 // CHUNK


def _sc_gather_body(xi_ref, ti_ref, ew_ref, eo_ref, g_ref, t_ref,
                    xi_v, ti_v, buf_v, sem_g):
    c = lax.axis_index("c")
    s = lax.axis_index("s")
    w = s * NC + c
    pltpu.sync_copy(xi_ref.at[pl.ds(w * CH_W, CH_W)], xi_v)
    pltpu.sync_copy(ti_ref.at[pl.ds(w * CH_W, CH_W)], ti_v)
    for tab_ref, idx_v, out_ref in (
        (ew_ref, xi_v, g_ref),
        (eo_ref, ti_v, t_ref),
    ):
        cps = [pltpu.async_copy(tab_ref.at[idx_v.at[j]], buf_v.at[j], sem_g)
               for j in range(CH_W)]
        for cp in cps:
            cp.wait()
        pltpu.sync_copy(buf_v, out_ref.at[pl.ds(w * CH_W, CH_W)])


@functools.cache
def _sc_gather():
    mesh = plsc.VectorSubcoreMesh(
        core_axis_name="c", subcore_axis_name="s", num_cores=NC, num_subcores=NS
    )
    return pl.kernel(
        _sc_gather_body,
        out_type=(
            jax.ShapeDtypeStruct((NW * CH_W, CHUNK, NDIM), jnp.float32),
            jax.ShapeDtypeStruct((NW * CH_W, CHUNK, NDIM), jnp.float32),
        ),
        mesh=mesh,
        scratch_types=(
            pltpu.VMEM((CH_W, CHUNK), jnp.int32),
            pltpu.VMEM((CH_W, CHUNK), jnp.int32),
            pltpu.VMEM((CH_W, CHUNK, NDIM), jnp.float32),
            pltpu.SemaphoreType.DMA,
        ),
        compiler_params=pltpu.CompilerParams(use_tc_tiling_on_sc=False),
    )


def _tc_dense(g_ref, t_ref, w_ref, b_ref, o_ref):
    g = g_ref[...]
    acc = jnp.dot(g, w_ref[...], preferred_element_type=jnp.float32)
    o = jnp.maximum(acc + b_ref[...], 0.0)
    a = jnp.sum(o * t_ref[...], axis=1)
    o_ref[...] = 1.0 / (1.0 + jnp.exp(-a))


_TC_BM = 2048

_tc_call = pl.pallas_call(
    _tc_dense,
    grid=(BATCH // _TC_BM,),
    in_specs=[
        pl.BlockSpec((_TC_BM, 128), lambda i: (i, 0)),
        pl.BlockSpec((_TC_BM, 128), lambda i: (i, 0)),
        pl.BlockSpec((128, 128), lambda i: (0, 0)),
        pl.BlockSpec((1, 128), lambda i: (0, 0)),
    ],
    out_specs=pl.BlockSpec((_TC_BM,), lambda i: (i,)),
    out_shape=jax.ShapeDtypeStruct((BATCH,), jnp.float32),
)


def kernel(x, target_id, embed_w, embed_out_w, fc1_w, fc1_b):
    xi = jnp.concatenate(
        [x, jnp.zeros((BATCH, 2), jnp.int32)], axis=1).reshape(-1, CHUNK)
    ti = jnp.concatenate(
        [target_id[:, None], jnp.zeros((BATCH, 7), jnp.int32)],
        axis=1).reshape(-1, CHUNK)

    graw, traw = _sc_gather()(xi, ti, embed_w, embed_out_w)
    g128 = graw.reshape(BATCH, 128)
    t128 = traw.reshape(BATCH, 128)

    w2 = jnp.zeros((128, 128), jnp.float32).at[:CTX * NDIM, :NDIM].set(fc1_w.T)
    b2 = jnp.zeros((1, 128), jnp.float32).at[0, :NDIM].set(fc1_b)
    return _tc_call(g128, t128, w2, b2)


# speedup vs baseline: 1.8422x; 1.8422x over previous
import functools

import jax
import jax.numpy as jnp
from jax import lax
from jax.experimental import pallas as pl
from jax.experimental.pallas import tpu as pltpu
from jax.experimental.pallas import tpu_sc as plsc

HORSE_COUNT = 1000000
NDIM = 16
BATCH = 16384
CTX = 6

NC = 2
NS = 16
NW = NC * NS

CHUNK = 128
ROWS_W = BATCH // NW
CH_W = ROWS_W * 8 // CHUNK


def _sc_gather_body(xi_ref, ti_ref, ew_ref, eo_ref, g_ref, t_ref,
                    xi_v, ti_v, buf_v, sem_g):
    c = lax.axis_index("c")
    s = lax.axis_index("s")
    w = s * NC + c
    pltpu.sync_copy(xi_ref.at[pl.ds(w * CH_W, CH_W)], xi_v)
    pltpu.sync_copy(ti_ref.at[pl.ds(w * CH_W, CH_W)], ti_v)
    for tab_ref, idx_v, out_ref in (
        (ew_ref, xi_v, g_ref),
        (eo_ref, ti_v, t_ref),
    ):
        cps = [pltpu.async_copy(tab_ref.at[idx_v.at[j]], buf_v.at[j], sem_g)
               for j in range(CH_W)]
        for cp in cps:
            cp.wait()
        pltpu.sync_copy(buf_v, out_ref.at[pl.ds(w * CH_W, CH_W)])


@functools.cache
def _sc_gather():
    mesh = plsc.VectorSubcoreMesh(
        core_axis_name="c", subcore_axis_name="s", num_cores=NC, num_subcores=NS
    )
    return pl.kernel(
        _sc_gather_body,
        out_type=(
            jax.ShapeDtypeStruct((NW * CH_W, CHUNK, NDIM), jnp.float32),
            jax.ShapeDtypeStruct((NW * CH_W, CHUNK, NDIM), jnp.float32),
        ),
        mesh=mesh,
        scratch_types=(
            pltpu.VMEM((CH_W, CHUNK), jnp.int32),
            pltpu.VMEM((CH_W, CHUNK), jnp.int32),
            pltpu.VMEM((CH_W, CHUNK, NDIM), jnp.float32),
            pltpu.SemaphoreType.DMA,
        ),
        compiler_params=pltpu.CompilerParams(use_tc_tiling_on_sc=False),
    )


def _tc_dense(g_ref, t_ref, w_ref, b_ref, o_ref):
    g = g_ref[...]
    acc = jnp.dot(g, w_ref[...], preferred_element_type=jnp.float32)
    o = jnp.maximum(acc + b_ref[...], 0.0)
    a = jnp.sum(o * t_ref[...], axis=1)
    o_ref[...] = 1.0 / (1.0 + jnp.exp(-a))


_TC_BM = 2048

_tc_call = pl.pallas_call(
    _tc_dense,
    grid=(BATCH // _TC_BM,),
    in_specs=[
        pl.BlockSpec((_TC_BM, 128), lambda i: (i, 0)),
        pl.BlockSpec((_TC_BM, 128), lambda i: (i, 0)),
        pl.BlockSpec((128, 128), lambda i: (0, 0)),
        pl.BlockSpec((1, 128), lambda i: (0, 0)),
    ],
    out_specs=pl.BlockSpec((_TC_BM,), lambda i: (i,)),
    out_shape=jax.ShapeDtypeStruct((BATCH,), jnp.float32),
)


def kernel(x, target_id, embed_w, embed_out_w, fc1_w, fc1_b):
    xpad = (jnp.arange(BATCH * 2, dtype=jnp.int32) % HORSE_COUNT
            ).reshape(BATCH, 2)
    tpad = (jnp.arange(BATCH * 7, dtype=jnp.int32) % HORSE_COUNT
            ).reshape(BATCH, 7)
    xi = jnp.concatenate([x, xpad], axis=1).reshape(-1, CHUNK)
    ti = jnp.concatenate([target_id[:, None], tpad], axis=1).reshape(-1, CHUNK)

    graw, traw = _sc_gather()(xi, ti, embed_w, embed_out_w)
    g128 = graw.reshape(BATCH, 128)
    t128 = traw.reshape(BATCH, 128)

    w2 = jnp.zeros((128, 128), jnp.float32).at[:CTX * NDIM, :NDIM].set(fc1_w.T)
    b2 = jnp.zeros((1, 128), jnp.float32).at[0, :NDIM].set(fc1_b)
    return _tc_call(g128, t128, w2, b2)
